# Initial kernel scaffold; baseline (speedup 1.0000x reference)
import jax
import jax.numpy as jnp
from jax.experimental import pallas as pl


def kernel(words_as_ids, hash_table, W, P):
    raise NotImplementedError("write your pallas kernel here")


# speedup vs baseline: 2.4783x; 2.4783x over previous
import jax
import jax.numpy as jnp
from jax import lax
from jax.experimental import pallas as pl
from jax.experimental.pallas import tpu as pltpu
from jax.experimental.pallas import tpu_sc as plsc

NUM_WORDS_K = 1000000
NUM_BUCKETS_K = 100000
EMB_K = 64
BATCH_K = 4096
SEQ_K = 200

NC = 2
NS = 16
LANES = 16
NW = NC * NS

N_TOK = BATCH_K * SEQ_K
TOK_PER_W = N_TOK // NW
CHUNK = 128
N_CHUNKS = TOK_PER_W // CHUNK
OUT_COLS = EMB_K + 2


def _sc_body(words, ht0, ht1, p0c, p1c, w_tab, out,
             wid_v, h0_v, h1_v, pa_v, pb_v, w0_v, w1_v, pv0_v, pv1_v,
             out_v, sem):
    wid = lax.axis_index("s") * NC + lax.axis_index("c")
    base_w = wid * TOK_PER_W

    iota = lax.iota(jnp.int32, LANES)

    def chunk_body(g, _):
        base = base_w + g * CHUNK
        pltpu.sync_copy(words.at[pl.ds(base, CHUNK)], wid_v)
        cs = [pltpu.async_copy(ht0.at[wid_v], h0_v, sem),
              pltpu.async_copy(ht1.at[wid_v], h1_v, sem),
              pltpu.async_copy(p0c.at[wid_v], pa_v, sem),
              pltpu.async_copy(p1c.at[wid_v], pb_v, sem)]
        for c in cs:
            c.wait()
        cs = [pltpu.async_copy(w_tab.at[h0_v], w0_v, sem),
              pltpu.async_copy(w_tab.at[h1_v], w1_v, sem),
              pltpu.async_copy(p0c.at[h0_v], pv0_v, sem),
              pltpu.async_copy(p1c.at[h1_v], pv1_v, sem)]
        for c in cs:
            c.wait()

        def tok_body(t, carry):
            tsplat = jnp.full((LANES,), t, jnp.int32)
            p0 = plsc.load_gather(pa_v, [tsplat])
            p1 = plsc.load_gather(pb_v, [tsplat])
            obase = t * OUT_COLS
            for j in range(EMB_K // LANES):
                sl = pl.ds(j * LANES, LANES)
                out_v[pl.ds(obase + j * LANES, LANES)] = (
                    w0_v[t, sl] * p0 + w1_v[t, sl] * p1)
            return carry

        lax.fori_loop(0, CHUNK, tok_body, None)

        def pv_body(g16, carry):
            rows = iota + g16 * LANES
            obase = rows * OUT_COLS
            pv0 = plsc.load_gather(pv0_v, [rows])
            pv1 = plsc.load_gather(pv1_v, [rows])
            plsc.store_scatter(out_v, [obase + EMB_K], pv0)
            plsc.store_scatter(out_v, [obase + EMB_K + 1], pv1)
            return carry

        lax.fori_loop(0, CHUNK // LANES, pv_body, None)

        pltpu.sync_copy(out_v, out.at[pl.ds(base * OUT_COLS, CHUNK * OUT_COLS)])
        return _

    lax.fori_loop(0, N_CHUNKS, chunk_body, None)


@jax.jit
def kernel(words_as_ids, hash_table, W, P):
    words_flat = words_as_ids.reshape(N_TOK).astype(jnp.int32)
    ht0 = hash_table[:, 0]
    ht1 = hash_table[:, 1]
    p0c = P[:, 0]
    p1c = P[:, 1]

    mesh = plsc.VectorSubcoreMesh(core_axis_name="c", subcore_axis_name="s",
                                  num_cores=NC, num_subcores=NS)
    out = pl.kernel(
        _sc_body,
        out_type=jax.ShapeDtypeStruct((N_TOK * OUT_COLS,), jnp.float32),
        mesh=mesh,
        compiler_params=pltpu.CompilerParams(
            needs_layout_passes=False, use_tc_tiling_on_sc=False),
        scratch_types=[
            pltpu.VMEM((CHUNK,), jnp.int32),
            pltpu.VMEM((CHUNK,), jnp.int32),
            pltpu.VMEM((CHUNK,), jnp.int32),
            pltpu.VMEM((CHUNK,), jnp.float32),
            pltpu.VMEM((CHUNK,), jnp.float32),
            pltpu.VMEM((CHUNK, EMB_K), jnp.float32),
            pltpu.VMEM((CHUNK, EMB_K), jnp.float32),
            pltpu.VMEM((CHUNK,), jnp.float32),
            pltpu.VMEM((CHUNK,), jnp.float32),
            pltpu.VMEM((CHUNK * OUT_COLS,), jnp.float32),
            pltpu.SemaphoreType.DMA,
        ],
    )(words_flat, ht0, ht1, p0c, p1c, W)
    return out.reshape(BATCH_K, SEQ_K, OUT_COLS)
